# baseline (speedup 1.0000x reference)
import functools

import jax
import jax.numpy as jnp
from jax import lax
from jax.experimental import pallas as pl
from jax.experimental.pallas import tpu as pltpu
from jax.experimental.pallas import tpu_sc as plsc

B = 1024
CTX = 20
EMB = 64
HID = 128
LANES = 16

_NC = 2
_NS = 16
_NW = _NC * _NS
_B_PER_W = B // _NW
_IDX_PER_W = _B_PER_W * CTX
_GCHUNK = 128
_NCHUNK = _IDX_PER_W // _GCHUNK


def _pool_body(emb_hbm, idx_hbm, out_hbm, idx_v, rows_v, pooled_v, sem):
    wid = lax.axis_index("s") * _NC + lax.axis_index("c")
    ibase = wid * _IDX_PER_W
    obase = wid * _B_PER_W

    pltpu.sync_copy(idx_hbm.at[pl.ds(ibase, _IDX_PER_W)], idx_v)

    copies = []
    for k in range(_NCHUNK):
        copies.append(pltpu.async_copy(
            emb_hbm.at[idx_v.at[pl.ds(k * _GCHUNK, _GCHUNK)]],
            rows_v.at[pl.ds(k * _GCHUNK, _GCHUNK)],
            sem,
        ))
    for c in copies:
        c.wait()

    def body(b, _):
        for ch in range(EMB // LANES):
            sl = pl.ds(ch * LANES, LANES)
            acc = rows_v[b * CTX, sl]
            for c in range(1, CTX):
                acc = acc + rows_v[b * CTX + c, sl]
            pooled_v[b, sl] = acc
        return _

    lax.fori_loop(0, _B_PER_W, body, None)
    pltpu.sync_copy(pooled_v, out_hbm.at[pl.ds(obase, _B_PER_W)])


_sc_pool = functools.partial(
    pl.kernel,
    mesh=plsc.VectorSubcoreMesh(core_axis_name="c", subcore_axis_name="s"),
    out_type=jax.ShapeDtypeStruct((B, EMB), jnp.float32),
    scratch_types=[
        pltpu.VMEM((_IDX_PER_W,), jnp.int32),
        pltpu.VMEM((_IDX_PER_W, EMB), jnp.float32),
        pltpu.VMEM((_B_PER_W, EMB), jnp.float32),
        pltpu.SemaphoreType.DMA,
    ],
    compiler_params=pltpu.CompilerParams(use_tc_tiling_on_sc=False),
)(_pool_body)


_VT = 512


def _mlp_body(pooled_ref, W1_ref, b1_ref, W2_ref, b2_ref, out_ref, h_ref):
    @pl.when(pl.program_id(0) == 0)
    def _():
        h = jnp.dot(pooled_ref[...], W1_ref[...],
                    preferred_element_type=jnp.float32)
        h_ref[...] = jnp.maximum(h + b1_ref[...], 0.0)

    out_ref[...] = jnp.dot(h_ref[...], W2_ref[...],
                           preferred_element_type=jnp.float32) + b2_ref[...]


def _tc_mlp(pooled, W1, b1, W2, b2):
    V = W2.shape[1]
    nv = pl.cdiv(V, _VT)
    return pl.pallas_call(
        _mlp_body,
        grid=(nv,),
        in_specs=[
            pl.BlockSpec((B, EMB), lambda i: (0, 0)),
            pl.BlockSpec((EMB, HID), lambda i: (0, 0)),
            pl.BlockSpec((1, HID), lambda i: (0, 0)),
            pl.BlockSpec((HID, _VT), lambda i: (0, i)),
            pl.BlockSpec((1, _VT), lambda i: (0, i)),
        ],
        out_specs=pl.BlockSpec((B, _VT), lambda i: (0, i)),
        out_shape=jax.ShapeDtypeStruct((B, V), jnp.float32),
        scratch_shapes=[pltpu.VMEM((B, HID), jnp.float32)],
        compiler_params=pltpu.CompilerParams(
            dimension_semantics=("arbitrary",)),
    )(pooled, W1, b1, W2, b2)


def kernel(inputs, emb, W1, b1, W2, b2):
    idx = inputs.reshape(-1).astype(jnp.int32)
    pooled = _sc_pool(emb, idx)
    return _tc_mlp(pooled, W1, b1.reshape(1, HID), W2, b2.reshape(1, -1))


# speedup vs baseline: 1.1336x; 1.1336x over previous
import functools

import jax
import jax.numpy as jnp
from jax import lax
from jax.experimental import pallas as pl
from jax.experimental.pallas import tpu as pltpu
from jax.experimental.pallas import tpu_sc as plsc

B = 1024
CTX = 20
EMB = 64
HID = 128
LANES = 16

_NC = 2
_NS = 16
_NW = _NC * _NS
_B_PER_W = B // _NW
_IDX_PER_W = _B_PER_W * CTX
_GCHUNK = 128
_NCHUNK = _IDX_PER_W // _GCHUNK


def _pool_body(emb_hbm, idx_hbm, out_hbm, idx_v, rows_v, pooled_v, sem):
    wid = lax.axis_index("s") * _NC + lax.axis_index("c")
    ibase = wid * _IDX_PER_W
    obase = wid * _B_PER_W

    pltpu.sync_copy(idx_hbm.at[pl.ds(ibase, _IDX_PER_W)], idx_v)

    copies = []
    for k in range(_NCHUNK):
        copies.append(pltpu.async_copy(
            emb_hbm.at[idx_v.at[pl.ds(k * _GCHUNK, _GCHUNK)]],
            rows_v.at[pl.ds(k * _GCHUNK, _GCHUNK)],
            sem,
        ))
    for c in copies:
        c.wait()

    def body(b, _):
        for ch in range(EMB // LANES):
            sl = pl.ds(ch * LANES, LANES)
            acc = rows_v[b * CTX, sl]
            for c in range(1, CTX):
                acc = acc + rows_v[b * CTX + c, sl]
            pooled_v[b, sl] = acc
        return _

    lax.fori_loop(0, _B_PER_W, body, None)
    pltpu.sync_copy(pooled_v, out_hbm.at[pl.ds(obase, _B_PER_W)])


_sc_pool = functools.partial(
    pl.kernel,
    mesh=plsc.VectorSubcoreMesh(core_axis_name="c", subcore_axis_name="s"),
    out_type=jax.ShapeDtypeStruct((B, EMB), jnp.float32),
    scratch_types=[
        pltpu.VMEM((_IDX_PER_W,), jnp.int32),
        pltpu.VMEM((_IDX_PER_W, EMB), jnp.float32),
        pltpu.VMEM((_B_PER_W, EMB), jnp.float32),
        pltpu.SemaphoreType.DMA,
    ],
    compiler_params=pltpu.CompilerParams(use_tc_tiling_on_sc=False),
)(_pool_body)


_VT = 4096


def _mlp_body(pooled_ref, W1_ref, b1_ref, W2_ref, b2_ref, out_ref, h_ref):
    @pl.when(pl.program_id(0) == 0)
    def _():
        h = jnp.dot(pooled_ref[...], W1_ref[...],
                    preferred_element_type=jnp.float32)
        h_ref[...] = jnp.maximum(h + b1_ref[...], 0.0)

    out_ref[...] = jnp.dot(h_ref[...], W2_ref[...],
                           preferred_element_type=jnp.float32) + b2_ref[...]


def _tc_mlp(pooled, W1, b1, W2, b2):
    V = W2.shape[1]
    nv = pl.cdiv(V, _VT)
    return pl.pallas_call(
        _mlp_body,
        grid=(nv,),
        in_specs=[
            pl.BlockSpec((B, EMB), lambda i: (0, 0)),
            pl.BlockSpec((EMB, HID), lambda i: (0, 0)),
            pl.BlockSpec((1, HID), lambda i: (0, 0)),
            pl.BlockSpec((HID, _VT), lambda i: (0, i)),
            pl.BlockSpec((1, _VT), lambda i: (0, i)),
        ],
        out_specs=pl.BlockSpec((B, _VT), lambda i: (0, i)),
        out_shape=jax.ShapeDtypeStruct((B, V), jnp.float32),
        scratch_shapes=[pltpu.VMEM((B, HID), jnp.float32)],
        compiler_params=pltpu.CompilerParams(
            dimension_semantics=("arbitrary",)),
    )(pooled, W1, b1, W2, b2)


def kernel(inputs, emb, W1, b1, W2, b2):
    idx = inputs.reshape(-1).astype(jnp.int32)
    pooled = _sc_pool(emb, idx)
    return _tc_mlp(pooled, W1, b1.reshape(1, HID), W2, b2.reshape(1, -1))
